# Initial kernel scaffold; baseline (speedup 1.0000x reference)
import jax
import jax.numpy as jnp
from jax.experimental import pallas as pl


def kernel(bbox_cls_pred, bbox_pred, bbox_dir_cls_pred, anchors):
    raise NotImplementedError("write your pallas kernel here")


# speedup vs baseline: 2.7673x; 2.7673x over previous
import numpy as np
import jax
import jax.numpy as jnp
from jax.experimental import pallas as pl
from jax.experimental.pallas import tpu as pltpu

_NCLS = 3
_NPRE = 100
_NMS_THR = 0.01
_SCORE_THR = 0.1
_MAX_NUM = 50
_YL, _XL = 248, 216
_YX = _YL * _XL
_PI = float(np.pi)
_F32 = jnp.float32


def _score_kernel(cls_ref, out_ref):
    out_ref[...] = jnp.max(cls_ref[...], axis=1)


def _sigmoid(x):
    return 1.0 / (1.0 + jnp.exp(-x))


def _decode_col(anc, dlt, dir_c):
    a0, a1, a2 = anc[:, 0:1], anc[:, 1:2], anc[:, 2:3]
    a3, a4, a5, a6 = anc[:, 3:4], anc[:, 4:5], anc[:, 5:6], anc[:, 6:7]
    da = jnp.sqrt(a3 * a3 + a4 * a4)
    x = dlt[:, 0:1] * da + a0
    y = dlt[:, 1:2] * da + a1
    z = dlt[:, 2:3] * a5 + a2 + a5 * 0.5
    w = a3 * jnp.exp(dlt[:, 3:4])
    l = a4 * jnp.exp(dlt[:, 4:5])
    h = a5 * jnp.exp(dlt[:, 5:6])
    z = z - h * 0.5
    th = a6 + dlt[:, 6:7]
    x1, y1 = x - w * 0.5, y - l * 0.5
    x2, y2 = x + w * 0.5, y + l * 0.5
    area = (x2 - x1) * (y2 - y1)
    lp = th - jnp.floor(th / _PI + 1.0) * _PI
    dirf = (dir_c[:, 1:2] > dir_c[:, 0:1]).astype(_F32)
    thf = lp + (1.0 - dirf) * _PI
    bb = jnp.concatenate([x, y, z, w, l, h, thf], axis=1)
    return bb, (x1, y1, x2, y2), area


def _decode_row(anc, dlt):
    a0, a1 = anc[0:1, :], anc[1:2, :]
    a3, a4 = anc[3:4, :], anc[4:5, :]
    da = jnp.sqrt(a3 * a3 + a4 * a4)
    x = dlt[0:1, :] * da + a0
    y = dlt[1:2, :] * da + a1
    w = a3 * jnp.exp(dlt[3:4, :])
    l = a4 * jnp.exp(dlt[4:5, :])
    x1, y1 = x - w * 0.5, y - l * 0.5
    x2, y2 = x + w * 0.5, y + l * 0.5
    area = (x2 - x1) * (y2 - y1)
    return (x1, y1, x2, y2), area


def _detect_kernel(cls_r, cls_c, box_r, box_c, dir_c, anc_r, anc_c,
                   boxes_out, labels_out, scores_out, iou_scr):
    f32 = _F32
    n = _NPRE
    bb_c, (x1c, y1c, x2c, y2c), area_c = _decode_col(
        anc_c[...], box_c[...], dir_c[...])
    (x1r, y1r, x2r, y2r), area_r = _decode_row(anc_r[...], box_r[...])

    wx = jnp.maximum(jnp.minimum(x2c, x2r) - jnp.maximum(x1c, x1r), 0.0)
    wy = jnp.maximum(jnp.minimum(y2c, y2r) - jnp.maximum(y1c, y1r), 0.0)
    inter = wx * wy
    union = jnp.maximum(area_c + area_r - inter, 1e-8)
    iou = inter / union

    sig_r = _sigmoid(cls_r[...])
    sig_c = _sigmoid(cls_c[...])

    it_col = jax.lax.broadcasted_iota(jnp.int32, (n, n), 0)
    it_row = jax.lax.broadcasted_iota(jnp.int32, (n, n), 1)
    lane_i = jax.lax.broadcasted_iota(jnp.int32, (1, n), 1)

    cls_scores = []
    for c in range(_NCLS):
        s_row = sig_r[c:c + 1, :]
        s_col = sig_c[:, c:c + 1]
        valid_r = (s_row > _SCORE_THR).astype(f32)
        key_r = jnp.where(s_row > _SCORE_THR, s_row, -1e30)
        key_c = jnp.where(s_col > _SCORE_THR, s_col, -1e30)
        gt = (key_c > key_r).astype(f32)
        eq = jnp.logical_and(key_c == key_r, it_col < it_row).astype(f32)
        rank = jnp.sum(gt + eq, axis=0, keepdims=True)
        perm = (it_col.astype(f32) == rank).astype(f32)
        tmp = jax.lax.dot_general(perm, iou, (((1,), (0,)), ((), ())),
                                  preferred_element_type=f32)
        iou_scr[...] = jax.lax.dot_general(tmp, perm, (((1,), (1,)), ((), ())),
                                           preferred_element_type=f32)
        v_sort = jax.lax.dot_general(valid_r, perm, (((1,), (1,)), ((), ())),
                                     preferred_element_type=f32)

        def nms_body(t, carry):
            sup, keep = carry
            oh = (lane_i == t).astype(f32)
            sup_t = jnp.sum(sup * oh)
            v_t = jnp.sum(v_sort * oh)
            ki = v_t * (1.0 - sup_t)
            keep = keep + ki * oh
            row = iou_scr[pl.ds(t, 1), :]
            sup = jnp.maximum(sup, ki * (row > _NMS_THR).astype(f32))
            return sup, keep

        zeros = jnp.zeros((1, n), f32)
        _, keep_s = jax.lax.fori_loop(0, n, nms_body, (zeros, zeros))
        keep_o = jax.lax.dot_general(keep_s, perm, (((1,), (0,)), ((), ())),
                                     preferred_element_type=f32)
        m = keep_o * valid_r
        cls_scores.append(jnp.where(m > 0.5, s_row, 0.0))

    s_all = jnp.concatenate(cls_scores, axis=0)
    flat_i = (jax.lax.broadcasted_iota(jnp.int32, (_NCLS, n), 0) * n
              + jax.lax.broadcasted_iota(jnp.int32, (_NCLS, n), 1))

    def top_body(k, sw):
        cur = jnp.max(sw)
        cand = jnp.where(sw == cur, flat_i, jnp.int32(2 ** 30))
        idx = jnp.min(cand)
        oh_flat = flat_i == idx
        oh100 = (lane_i == (idx % n)).astype(f32)
        brow = jax.lax.dot_general(oh100, bb_c, (((1,), (0,)), ((), ())),
                                   preferred_element_type=f32)
        boxes_out[pl.ds(k, 1), :] = brow
        labels_out[pl.ds(k, 1), :] = jnp.reshape(idx // n, (1, 1))
        scores_out[pl.ds(k, 1), :] = jnp.reshape(cur, (1, 1))
        return jnp.where(oh_flat, -1e30, sw)

    jax.lax.fori_loop(0, _MAX_NUM, top_body, s_all)


@jax.jit
def kernel(bbox_cls_pred, bbox_pred, bbox_dir_cls_pred, anchors):
    f32 = _F32
    cls2d = bbox_cls_pred.reshape(18, _YX)
    box2d = bbox_pred.reshape(42, _YX)
    dir2d = bbox_dir_cls_pred.reshape(12, _YX)

    m6 = pl.pallas_call(
        _score_kernel,
        out_shape=jax.ShapeDtypeStruct((6, _YX), f32),
    )(cls2d.reshape(6, 3, _YX))

    maxs = m6.T.reshape(-1)
    _, flat = jax.lax.top_k(maxs, _NPRE)
    a = flat % 6
    p = flat // 6

    cls100 = cls2d[3 * a[:, None] + jnp.arange(3)[None, :], p[:, None]]
    box100 = box2d[7 * a[:, None] + jnp.arange(7)[None, :], p[:, None]]
    dir100 = dir2d[2 * a[:, None] + jnp.arange(2)[None, :], p[:, None]]
    anc100 = anchors.reshape(-1, 7)[flat]

    boxes, labels, scores = pl.pallas_call(
        _detect_kernel,
        out_shape=(
            jax.ShapeDtypeStruct((_MAX_NUM, 7), f32),
            jax.ShapeDtypeStruct((_MAX_NUM, 1), jnp.int32),
            jax.ShapeDtypeStruct((_MAX_NUM, 1), f32),
        ),
        scratch_shapes=[pltpu.VMEM((_NPRE, _NPRE), f32)],
    )(cls100.T, cls100, box100.T, box100, dir100, anc100.T, anc100)

    return boxes, labels.reshape(-1), scores.reshape(-1)


# speedup vs baseline: 2.9109x; 1.0519x over previous
import numpy as np
import jax
import jax.numpy as jnp
from jax.experimental import pallas as pl
from jax.experimental.pallas import tpu as pltpu

_NCLS = 3
_NPRE = 100
_NMS_THR = 0.01
_SCORE_THR = 0.1
_MAX_NUM = 50
_YL, _XL = 248, 216
_YX = _YL * _XL
_PI = float(np.pi)
_F32 = jnp.float32


def _score_kernel(cls_ref, out_ref):
    out_ref[...] = jnp.max(cls_ref[...], axis=1)


def _sigmoid(x):
    return 1.0 / (1.0 + jnp.exp(-x))


def _decode_col(anc, dlt, dir_c):
    a0, a1, a2 = anc[:, 0:1], anc[:, 1:2], anc[:, 2:3]
    a3, a4, a5, a6 = anc[:, 3:4], anc[:, 4:5], anc[:, 5:6], anc[:, 6:7]
    da = jnp.sqrt(a3 * a3 + a4 * a4)
    x = dlt[:, 0:1] * da + a0
    y = dlt[:, 1:2] * da + a1
    z = dlt[:, 2:3] * a5 + a2 + a5 * 0.5
    w = a3 * jnp.exp(dlt[:, 3:4])
    l = a4 * jnp.exp(dlt[:, 4:5])
    h = a5 * jnp.exp(dlt[:, 5:6])
    z = z - h * 0.5
    th = a6 + dlt[:, 6:7]
    x1, y1 = x - w * 0.5, y - l * 0.5
    x2, y2 = x + w * 0.5, y + l * 0.5
    area = (x2 - x1) * (y2 - y1)
    lp = th - jnp.floor(th / _PI + 1.0) * _PI
    dirf = (dir_c[:, 1:2] > dir_c[:, 0:1]).astype(_F32)
    thf = lp + (1.0 - dirf) * _PI
    bb = jnp.concatenate([x, y, z, w, l, h, thf], axis=1)
    return bb, (x1, y1, x2, y2), area


def _decode_row(anc, dlt):
    a0, a1 = anc[0:1, :], anc[1:2, :]
    a3, a4 = anc[3:4, :], anc[4:5, :]
    da = jnp.sqrt(a3 * a3 + a4 * a4)
    x = dlt[0:1, :] * da + a0
    y = dlt[1:2, :] * da + a1
    w = a3 * jnp.exp(dlt[3:4, :])
    l = a4 * jnp.exp(dlt[4:5, :])
    x1, y1 = x - w * 0.5, y - l * 0.5
    x2, y2 = x + w * 0.5, y + l * 0.5
    area = (x2 - x1) * (y2 - y1)
    return (x1, y1, x2, y2), area


def _detect_kernel(cls_r, cls_c, box_r, box_c, dir_c, anc_r, anc_c,
                   boxes_out, labels_out, scores_out, iou_scr):
    f32 = _F32
    n = _NPRE
    bb_c, (x1c, y1c, x2c, y2c), area_c = _decode_col(
        anc_c[...], box_c[...], dir_c[...])
    (x1r, y1r, x2r, y2r), area_r = _decode_row(anc_r[...], box_r[...])

    wx = jnp.maximum(jnp.minimum(x2c, x2r) - jnp.maximum(x1c, x1r), 0.0)
    wy = jnp.maximum(jnp.minimum(y2c, y2r) - jnp.maximum(y1c, y1r), 0.0)
    inter = wx * wy
    union = jnp.maximum(area_c + area_r - inter, 1e-8)
    iou = inter / union

    sig_r = _sigmoid(cls_r[...])
    sig_c = _sigmoid(cls_c[...])

    it_col = jax.lax.broadcasted_iota(jnp.int32, (n, n), 0)
    it_row = jax.lax.broadcasted_iota(jnp.int32, (n, n), 1)
    lane_i = jax.lax.broadcasted_iota(jnp.int32, (1, n), 1)

    cls_scores = []
    for c in range(_NCLS):
        s_row = sig_r[c:c + 1, :]
        s_col = sig_c[:, c:c + 1]
        valid_r = (s_row > _SCORE_THR).astype(f32)
        key_r = jnp.where(s_row > _SCORE_THR, s_row, -1e30)
        key_c = jnp.where(s_col > _SCORE_THR, s_col, -1e30)
        gt = (key_c > key_r).astype(f32)
        eq = jnp.logical_and(key_c == key_r, it_col < it_row).astype(f32)
        rank = jnp.sum(gt + eq, axis=0, keepdims=True)
        perm = (it_col.astype(f32) == rank).astype(f32)
        tmp = jax.lax.dot_general(perm, iou, (((1,), (0,)), ((), ())),
                                  preferred_element_type=f32)
        iou_scr[...] = jax.lax.dot_general(tmp, perm, (((1,), (1,)), ((), ())),
                                           preferred_element_type=f32)
        v_sort = jax.lax.dot_general(valid_r, perm, (((1,), (1,)), ((), ())),
                                     preferred_element_type=f32)

        def nms_body(t, carry):
            sup, keep = carry
            oh = (lane_i == t).astype(f32)
            sup_t = jnp.sum(sup * oh)
            v_t = jnp.sum(v_sort * oh)
            ki = v_t * (1.0 - sup_t)
            keep = keep + ki * oh
            row = iou_scr[pl.ds(t, 1), :]
            sup = jnp.maximum(sup, ki * (row > _NMS_THR).astype(f32))
            return sup, keep

        zeros = jnp.zeros((1, n), f32)
        _, keep_s = jax.lax.fori_loop(0, n, nms_body, (zeros, zeros))
        keep_o = jax.lax.dot_general(keep_s, perm, (((1,), (0,)), ((), ())),
                                     preferred_element_type=f32)
        m = keep_o * valid_r
        cls_scores.append(jnp.where(m > 0.5, s_row, 0.0))

    s_all = jnp.concatenate(cls_scores, axis=0)
    flat_i = (jax.lax.broadcasted_iota(jnp.int32, (_NCLS, n), 0) * n
              + jax.lax.broadcasted_iota(jnp.int32, (_NCLS, n), 1))

    def top_body(k, sw):
        cur = jnp.max(sw)
        cand = jnp.where(sw == cur, flat_i, jnp.int32(2 ** 30))
        idx = jnp.min(cand)
        oh_flat = flat_i == idx
        oh100 = (lane_i == (idx % n)).astype(f32)
        brow = jax.lax.dot_general(oh100, bb_c, (((1,), (0,)), ((), ())),
                                   preferred_element_type=f32)
        boxes_out[pl.ds(k, 1), :] = brow
        labels_out[pl.ds(k, 1), :] = jnp.reshape(idx // n, (1, 1))
        scores_out[pl.ds(k, 1), :] = jnp.reshape(cur, (1, 1))
        return jnp.where(oh_flat, -1e30, sw)

    jax.lax.fori_loop(0, _MAX_NUM, top_body, s_all)


@jax.jit
def kernel(bbox_cls_pred, bbox_pred, bbox_dir_cls_pred, anchors):
    f32 = _F32
    cls2d = bbox_cls_pred.reshape(18, _YX)
    box2d = bbox_pred.reshape(42, _YX)
    dir2d = bbox_dir_cls_pred.reshape(12, _YX)

    m6 = pl.pallas_call(
        _score_kernel,
        out_shape=jax.ShapeDtypeStruct((6, _YX), f32),
    )(cls2d.reshape(6, 3, _YX))

    vals, idx_t = jax.lax.top_k(m6.reshape(-1), _NPRE)
    a_t = idx_t // _YX
    p_t = idx_t % _YX
    pflat_t = p_t * 6 + a_t
    _, flat, a, p = jax.lax.sort((-vals, pflat_t, a_t, p_t), num_keys=2)

    cls100 = cls2d.reshape(-1)[(3 * a[:, None] + jnp.arange(3)[None, :])
                               * _YX + p[:, None]]
    box100 = box2d.reshape(-1)[(7 * a[:, None] + jnp.arange(7)[None, :])
                               * _YX + p[:, None]]
    dir100 = dir2d.reshape(-1)[(2 * a[:, None] + jnp.arange(2)[None, :])
                               * _YX + p[:, None]]
    anc100 = anchors.reshape(-1, 7)[flat]

    boxes, labels, scores = pl.pallas_call(
        _detect_kernel,
        out_shape=(
            jax.ShapeDtypeStruct((_MAX_NUM, 7), f32),
            jax.ShapeDtypeStruct((_MAX_NUM, 1), jnp.int32),
            jax.ShapeDtypeStruct((_MAX_NUM, 1), f32),
        ),
        scratch_shapes=[pltpu.VMEM((_NPRE, _NPRE), f32)],
    )(cls100.T, cls100, box100.T, box100, dir100, anc100.T, anc100)

    return boxes, labels.reshape(-1), scores.reshape(-1)


# speedup vs baseline: 3.1277x; 1.0745x over previous
import numpy as np
import jax
import jax.numpy as jnp
from jax.experimental import pallas as pl
from jax.experimental.pallas import tpu as pltpu

_NCLS = 3
_NPRE = 100
_NMS_THR = 0.01
_SCORE_THR = 0.1
_MAX_NUM = 50
_YL, _XL = 248, 216
_YX = _YL * _XL
_PI = float(np.pi)
_F32 = jnp.float32


def _score_kernel(cls_ref, out_ref):
    for a in range(6):
        out_ref[a] = jnp.maximum(
            jnp.maximum(cls_ref[3 * a], cls_ref[3 * a + 1]),
            cls_ref[3 * a + 2])


def _sigmoid(x):
    return 1.0 / (1.0 + jnp.exp(-x))


def _decode_col(anc, dlt, dir_c):
    a0, a1, a2 = anc[:, 0:1], anc[:, 1:2], anc[:, 2:3]
    a3, a4, a5, a6 = anc[:, 3:4], anc[:, 4:5], anc[:, 5:6], anc[:, 6:7]
    da = jnp.sqrt(a3 * a3 + a4 * a4)
    x = dlt[:, 0:1] * da + a0
    y = dlt[:, 1:2] * da + a1
    z = dlt[:, 2:3] * a5 + a2 + a5 * 0.5
    w = a3 * jnp.exp(dlt[:, 3:4])
    l = a4 * jnp.exp(dlt[:, 4:5])
    h = a5 * jnp.exp(dlt[:, 5:6])
    z = z - h * 0.5
    th = a6 + dlt[:, 6:7]
    x1, y1 = x - w * 0.5, y - l * 0.5
    x2, y2 = x + w * 0.5, y + l * 0.5
    area = (x2 - x1) * (y2 - y1)
    lp = th - jnp.floor(th / _PI + 1.0) * _PI
    dirf = (dir_c[:, 1:2] > dir_c[:, 0:1]).astype(_F32)
    thf = lp + (1.0 - dirf) * _PI
    bb = jnp.concatenate([x, y, z, w, l, h, thf], axis=1)
    return bb, (x1, y1, x2, y2), area


def _decode_row(anc, dlt):
    a0, a1 = anc[0:1, :], anc[1:2, :]
    a3, a4 = anc[3:4, :], anc[4:5, :]
    da = jnp.sqrt(a3 * a3 + a4 * a4)
    x = dlt[0:1, :] * da + a0
    y = dlt[1:2, :] * da + a1
    w = a3 * jnp.exp(dlt[3:4, :])
    l = a4 * jnp.exp(dlt[4:5, :])
    x1, y1 = x - w * 0.5, y - l * 0.5
    x2, y2 = x + w * 0.5, y + l * 0.5
    area = (x2 - x1) * (y2 - y1)
    return (x1, y1, x2, y2), area


def _detect_kernel(cls_r, cls_c, box_r, box_c, dir_c, anc_r, anc_c,
                   boxes_out, labels_out, scores_out, iou_scr):
    f32 = _F32
    n = _NPRE
    bb_c, (x1c, y1c, x2c, y2c), area_c = _decode_col(
        anc_c[...], box_c[...], dir_c[...])
    (x1r, y1r, x2r, y2r), area_r = _decode_row(anc_r[...], box_r[...])

    wx = jnp.maximum(jnp.minimum(x2c, x2r) - jnp.maximum(x1c, x1r), 0.0)
    wy = jnp.maximum(jnp.minimum(y2c, y2r) - jnp.maximum(y1c, y1r), 0.0)
    inter = wx * wy
    union = jnp.maximum(area_c + area_r - inter, 1e-8)
    iou = inter / union

    sig_r = _sigmoid(cls_r[...])
    sig_c = _sigmoid(cls_c[...])

    it_col = jax.lax.broadcasted_iota(jnp.int32, (n, n), 0)
    it_row = jax.lax.broadcasted_iota(jnp.int32, (n, n), 1)
    lane_i = jax.lax.broadcasted_iota(jnp.int32, (1, n), 1)

    cls_scores = []
    for c in range(_NCLS):
        s_row = sig_r[c:c + 1, :]
        s_col = sig_c[:, c:c + 1]
        valid_r = (s_row > _SCORE_THR).astype(f32)
        key_r = jnp.where(s_row > _SCORE_THR, s_row, -1e30)
        key_c = jnp.where(s_col > _SCORE_THR, s_col, -1e30)
        gt = (key_c > key_r).astype(f32)
        eq = jnp.logical_and(key_c == key_r, it_col < it_row).astype(f32)
        rank = jnp.sum(gt + eq, axis=0, keepdims=True)
        perm = (it_col.astype(f32) == rank).astype(f32)
        tmp = jax.lax.dot_general(perm, iou, (((1,), (0,)), ((), ())),
                                  preferred_element_type=f32)
        iou_scr[...] = jax.lax.dot_general(tmp, perm, (((1,), (1,)), ((), ())),
                                           preferred_element_type=f32)
        v_sort = jax.lax.dot_general(valid_r, perm, (((1,), (1,)), ((), ())),
                                     preferred_element_type=f32)

        def nms_body(t, carry):
            sup, keep = carry
            oh = (lane_i == t).astype(f32)
            sup_t = jnp.sum(sup * oh)
            v_t = jnp.sum(v_sort * oh)
            ki = v_t * (1.0 - sup_t)
            keep = keep + ki * oh
            row = iou_scr[pl.ds(t, 1), :]
            sup = jnp.maximum(sup, ki * (row > _NMS_THR).astype(f32))
            return sup, keep

        zeros = jnp.zeros((1, n), f32)
        _, keep_s = jax.lax.fori_loop(0, n, nms_body, (zeros, zeros))
        keep_o = jax.lax.dot_general(keep_s, perm, (((1,), (0,)), ((), ())),
                                     preferred_element_type=f32)
        m = keep_o * valid_r
        cls_scores.append(jnp.where(m > 0.5, s_row, 0.0))

    s_all = jnp.concatenate(cls_scores, axis=0)
    flat_i = (jax.lax.broadcasted_iota(jnp.int32, (_NCLS, n), 0) * n
              + jax.lax.broadcasted_iota(jnp.int32, (_NCLS, n), 1))

    def top_body(k, sw):
        cur = jnp.max(sw)
        cand = jnp.where(sw == cur, flat_i, jnp.int32(2 ** 30))
        idx = jnp.min(cand)
        oh_flat = flat_i == idx
        oh100 = (lane_i == (idx % n)).astype(f32)
        brow = jax.lax.dot_general(oh100, bb_c, (((1,), (0,)), ((), ())),
                                   preferred_element_type=f32)
        boxes_out[pl.ds(k, 1), :] = brow
        labels_out[pl.ds(k, 1), :] = jnp.reshape(idx // n, (1, 1))
        scores_out[pl.ds(k, 1), :] = jnp.reshape(cur, (1, 1))
        return jnp.where(oh_flat, -1e30, sw)

    jax.lax.fori_loop(0, _MAX_NUM, top_body, s_all)


@jax.jit
def kernel(bbox_cls_pred, bbox_pred, bbox_dir_cls_pred, anchors):
    f32 = _F32

    m6 = pl.pallas_call(
        _score_kernel,
        out_shape=jax.ShapeDtypeStruct((6, _YL, _XL), f32),
    )(bbox_cls_pred)

    vals, idx_t = jax.lax.top_k(m6.reshape(-1), _NPRE)
    a_t = idx_t // _YX
    p_t = idx_t % _YX
    pflat_t = p_t * 6 + a_t
    _, flat, a, p = jax.lax.sort((-vals, pflat_t, a_t, p_t), num_keys=2)
    yy = (p // _XL)[:, None]
    xx = (p % _XL)[:, None]

    cls100 = bbox_cls_pred[3 * a[:, None] + jnp.arange(3)[None, :], yy, xx]
    box100 = bbox_pred[7 * a[:, None] + jnp.arange(7)[None, :], yy, xx]
    dir100 = bbox_dir_cls_pred[2 * a[:, None] + jnp.arange(2)[None, :], yy, xx]
    anc100 = anchors[yy, xx, (a // 2)[:, None], (a % 2)[:, None],
                     jnp.arange(7)[None, :]]

    boxes, labels, scores = pl.pallas_call(
        _detect_kernel,
        out_shape=(
            jax.ShapeDtypeStruct((_MAX_NUM, 7), f32),
            jax.ShapeDtypeStruct((_MAX_NUM, 1), jnp.int32),
            jax.ShapeDtypeStruct((_MAX_NUM, 1), f32),
        ),
        scratch_shapes=[pltpu.VMEM((_NPRE, _NPRE), f32)],
    )(cls100.T, cls100, box100.T, box100, dir100, anc100.T, anc100)

    return boxes, labels.reshape(-1), scores.reshape(-1)
